# baseline (speedup 1.0000x reference)
import functools

import jax
import jax.numpy as jnp
from jax.experimental import pallas as pl

K_NN = 10


def _body(xb_ref, xa_ref, h_ref, *, n_rows, blk):
    xb = xb_ref[...]
    xa = xa_ref[...]
    d = xb.shape[1]

    sqa = jnp.sum(xa * xa, axis=1, keepdims=True)
    sqb_row = jax.lax.dot_general(
        jnp.ones((1, d), jnp.float32), xb * xb,
        (((1,), (1,)), ((), ())), precision=jax.lax.Precision.HIGHEST,
        preferred_element_type=jnp.float32)
    mm = jax.lax.dot_general(
        xa, xb, (((1,), (1,)), ((), ())),
        preferred_element_type=jnp.float32)
    dist = jnp.maximum(sqa + sqb_row - 2.0 * mm, 0.0)

    avg = jnp.sum(dist, axis=0, keepdims=True) * (1.0 / n_rows)

    rows = jax.lax.broadcasted_iota(jnp.int32, dist.shape, 0)
    intmax = jnp.int32(2**31 - 1)
    inf = jnp.float32(jnp.inf)
    p = dist
    selmask = jnp.zeros(dist.shape, jnp.float32)
    for _ in range(K_NN):
        m = jnp.min(p, axis=0, keepdims=True)
        t = jnp.where(p == m, rows, intmax)
        j = jnp.min(t, axis=0, keepdims=True)
        hit = rows == j
        selmask = selmask + jnp.where(hit, 1.0, 0.0)
        p = jnp.where(hit, inf, p)

    inv = 1.0 / (avg * avg + 1e-12)
    h_ref[...] = selmask * jnp.exp(-(dist * dist) * inv)


def kernel(inputs):
    x = inputs
    n, d = x.shape
    blk = 128
    grid = n // blk
    body = functools.partial(_body, n_rows=n, blk=blk)
    return pl.pallas_call(
        body,
        grid=(grid,),
        in_specs=[
            pl.BlockSpec((blk, d), lambda i: (i, 0)),
            pl.BlockSpec((n, d), lambda i: (0, 0)),
        ],
        out_specs=pl.BlockSpec((n, blk), lambda i: (0, i)),
        out_shape=jax.ShapeDtypeStruct((n, n), jnp.float32),
    )(x, x)


# speedup vs baseline: 1.0565x; 1.0565x over previous
import functools

import jax
import jax.numpy as jnp
from jax.experimental import pallas as pl

K_NN = 10


def _body(xb_ref, xa_ref, h_ref, *, n_rows, blk):
    xb = xb_ref[...]
    xa = xa_ref[...]
    d = xb.shape[1]

    sqa = jnp.sum(xa * xa, axis=1, keepdims=True)
    sqb_row = jax.lax.dot_general(
        jnp.ones((1, d), jnp.float32), xb * xb,
        (((1,), (1,)), ((), ())), precision=jax.lax.Precision.HIGHEST,
        preferred_element_type=jnp.float32)
    mm = jax.lax.dot_general(
        xa, xb, (((1,), (1,)), ((), ())),
        preferred_element_type=jnp.float32)
    dist = jnp.maximum(sqa + sqb_row - 2.0 * mm, 0.0)

    avg = jnp.sum(dist, axis=0, keepdims=True) * (1.0 / n_rows)

    rows = jax.lax.broadcasted_iota(jnp.int32, dist.shape, 0)
    intmax = jnp.int32(2**31 - 1)
    inf = jnp.float32(jnp.inf)
    m = jnp.min(dist, axis=0, keepdims=True)
    j = jnp.min(jnp.where(dist == m, rows, intmax), axis=0, keepdims=True)
    for _ in range(K_NN - 1):
        done = (dist < m) | ((dist == m) & (rows <= j))
        contrib = jnp.where(done, inf, dist)
        m = jnp.min(contrib, axis=0, keepdims=True)
        j = jnp.min(jnp.where(contrib == m, rows, intmax), axis=0,
                    keepdims=True)

    inv = 1.0 / (avg * avg + 1e-12)
    mask = (dist < m) | ((dist == m) & (rows <= j))
    h_ref[...] = jnp.where(mask, jnp.exp(-(dist * dist) * inv), 0.0)


def kernel(inputs):
    x = inputs
    n, d = x.shape
    blk = 256
    grid = n // blk
    body = functools.partial(_body, n_rows=n, blk=blk)
    return pl.pallas_call(
        body,
        grid=(grid,),
        in_specs=[
            pl.BlockSpec((blk, d), lambda i: (i, 0)),
            pl.BlockSpec((n, d), lambda i: (0, 0)),
        ],
        out_specs=pl.BlockSpec((n, blk), lambda i: (0, i)),
        out_shape=jax.ShapeDtypeStruct((n, n), jnp.float32),
    )(x, x)


# speedup vs baseline: 1.9340x; 1.8307x over previous
import functools

import jax
import jax.numpy as jnp
from jax.experimental import pallas as pl

K_NN = 10


def _body(xb_ref, xa_ref, h_ref, *, n_rows, blk):
    xb = xb_ref[...]
    xa = xa_ref[...]
    d = xb.shape[1]

    sqa = jnp.sum(xa * xa, axis=1, keepdims=True)
    sqb_row = jax.lax.dot_general(
        jnp.ones((1, d), jnp.float32), xb * xb,
        (((1,), (1,)), ((), ())), precision=jax.lax.Precision.HIGHEST,
        preferred_element_type=jnp.float32)
    mm = jax.lax.dot_general(
        xa, xb, (((1,), (1,)), ((), ())),
        preferred_element_type=jnp.float32)
    dist = jnp.maximum(sqa + sqb_row - 2.0 * mm, 0.0)

    avg = jnp.sum(dist, axis=0, keepdims=True) * (1.0 / n_rows)

    rows = jax.lax.broadcasted_iota(jnp.int32, dist.shape, 0)
    hi = jnp.int32(-8192)
    lo = jnp.int32(8191)
    intmax = jnp.int32(2**31 - 1)
    inf32 = jnp.float32(jnp.inf)

    bits = jax.lax.bitcast_convert_type(dist, jnp.int32)
    pk = (bits & hi) | rows

    m = jnp.min(pk, axis=0, keepdims=True)
    for _ in range(K_NN - 1):
        m = jnp.min(jnp.where(pk > m, pk, intmax), axis=0, keepdims=True)
    t_b = m & hi

    trunc = pk & hi
    in_bucket = trunc == t_b
    nless = jnp.sum(jnp.where(trunc < t_b, 1, 0), axis=0, keepdims=True)
    nbucket = jnp.sum(jnp.where(in_bucket, 1, 0), axis=0, keepdims=True)
    take = K_NN - nless

    need = nbucket > take
    vb0 = jnp.where(need, -inf32, inf32)
    jb0 = jnp.where(need, jnp.int32(-1), intmax)
    cnt0 = jnp.where(need, 0, take)

    def _cond(carry):
        _, _, cnt = carry
        return jnp.any(cnt < take)

    def _refine(carry):
        vb, jb, cnt = carry
        act = cnt < take
        lexgt = (dist > vb) | ((dist == vb) & (rows > jb))
        contrib = jnp.where(in_bucket & lexgt, dist, inf32)
        m2 = jnp.min(contrib, axis=0, keepdims=True)
        j2 = jnp.min(jnp.where(contrib == m2, rows, intmax), axis=0,
                     keepdims=True)
        vb = jnp.where(act, m2, vb)
        jb = jnp.where(act, j2, jb)
        return vb, jb, cnt + jnp.where(act, 1, 0)

    vb, jb, _ = jax.lax.while_loop(_cond, _refine, (vb0, jb0, cnt0))

    mask = (trunc < t_b) | (in_bucket &
                            ((dist < vb) | ((dist == vb) & (rows <= jb))))
    inv = 1.0 / (avg * avg + 1e-12)
    h_ref[...] = jnp.where(mask, jnp.exp(-(dist * dist) * inv), 0.0)


def kernel(inputs):
    x = inputs
    n, d = x.shape
    blk = 256
    grid = n // blk
    body = functools.partial(_body, n_rows=n, blk=blk)
    return pl.pallas_call(
        body,
        grid=(grid,),
        in_specs=[
            pl.BlockSpec((blk, d), lambda i: (i, 0)),
            pl.BlockSpec((n, d), lambda i: (0, 0)),
        ],
        out_specs=pl.BlockSpec((n, blk), lambda i: (0, i)),
        out_shape=jax.ShapeDtypeStruct((n, n), jnp.float32),
    )(x, x)
